# baseline (speedup 1.0000x reference)
import functools

import jax
import jax.numpy as jnp
from jax import lax
from jax.experimental import pallas as pl
from jax.experimental.pallas import tpu as pltpu
from jax.experimental.pallas import tpu_sc as plsc

B = 4
S = 2048
D = 1024
NSPAN = 128
SMAX = 64
EROWS = 2 * SMAX
L = 16
NC = 2
NS = 16
NW = NC * NS
DB = 128
NDB = D // DB

_mesh = plsc.VectorSubcoreMesh(core_axis_name="c", subcore_axis_name="s")


@functools.partial(
    pl.kernel,
    out_type=jax.ShapeDtypeStruct((B, NSPAN, D), jnp.float32),
    mesh=_mesh,
    compiler_params=pltpu.CompilerParams(
        needs_layout_passes=False, skip_device_barrier=True),
    scratch_types=[
        pltpu.VMEM((SMAX, DB), jnp.float32),
        pltpu.VMEM((EROWS, DB), jnp.float32),
        pltpu.VMEM((B * NSPAN * 2,), jnp.int32),
        pltpu.VMEM((NSPAN, DB), jnp.float32),
        pltpu.VMEM((NSPAN + L,), jnp.int32),
        pltpu.VMEM((NSPAN + L,), jnp.int32),
        pltpu.VMEM((NSPAN + L,), jnp.float32),
        pltpu.SemaphoreType.DMA,
    ],
)
def _span_avg(seq_hbm, sp_hbm, out_hbm, s_v, e_v, sp_v, o_v,
              hi_v, lo_v, iw_v, sem):
    wid = lax.axis_index("s") * NC + lax.axis_index("c")
    b = wid // NDB
    d0 = (wid % NDB) * DB
    seq_cp = pltpu.async_copy(
        seq_hbm.at[b, pl.ds(0, SMAX), pl.ds(d0, DB)], s_v, sem)
    pltpu.sync_copy(sp_hbm, sp_v)

    lanes = lax.iota(jnp.int32, L)

    waccs = [jnp.zeros((L,), jnp.int32) for _ in range(4)]
    for c in range(B * NSPAN // L):
        f = c * L + lanes
        sv = plsc.load_gather(sp_v, [2 * f])
        ev = plsc.load_gather(sp_v, [2 * f + 1])
        waccs[c % 4] = jnp.maximum(waccs[c % 4], ev - sv)
    w_glob = jnp.max(jnp.maximum(jnp.maximum(waccs[0], waccs[1]),
                                 jnp.maximum(waccs[2], waccs[3])))

    for c in range(NSPAN // L):
        f = (b * NSPAN + c * L) + lanes
        sv = plsc.load_gather(sp_v, [2 * f])
        ev = plsc.load_gather(sp_v, [2 * f + 1])
        weff = jnp.where(sv < ev, ev - sv, w_glob)
        hi = ev + SMAX
        lo = hi - weff
        invw = jnp.where(weff > 0, 1.0, 0.0) / jnp.maximum(
            weff, 1).astype(jnp.float32)
        sl = pl.ds(c * L, L)
        hi_v[sl] = hi
        lo_v[sl] = lo
        iw_v[sl] = invw

    seq_cp.wait()

    s0s = tuple(s_v[0, pl.ds(ch * L, L)] for ch in range(DB // L))
    zeros8 = tuple(jnp.zeros((L,), jnp.float32) for _ in range(DB // L))

    @plsc.parallel_loop(0, SMAX, step=1, unroll=4, carry=zeros8)
    def mids(m, accs):
        for ch in range(DB // L):
            e_v[m, pl.ds(ch * L, L)] = accs[ch]
        return tuple(a + s0 for a, s0 in zip(accs, s0s))

    @plsc.parallel_loop(0, SMAX, step=1, unroll=4, carry=mids)
    def _(k, accs):
        for ch in range(DB // L):
            e_v[SMAX + k, pl.ds(ch * L, L)] = accs[ch]
        return tuple(a + s_v[k, pl.ds(ch * L, L)]
                     for ch, a in enumerate(accs))

    def span_half(lo_i, hi_i):
        @plsc.parallel_loop(lo_i, hi_i, step=1, unroll=8)
        def _(i):
            hi_j = hi_v[pl.ds(i, L)][0]
            lo_j = lo_v[pl.ds(i, L)][0]
            iw_j = iw_v[pl.ds(i, L)][0]
            for ch in range(DB // L):
                sl = pl.ds(ch * L, L)
                o_v[i, sl] = (e_v[hi_j, sl] - e_v[lo_j, sl]) * iw_j
            return ()

    span_half(0, NSPAN // 2)
    out_cp0 = pltpu.async_copy(
        o_v.at[pl.ds(0, NSPAN // 2), :],
        out_hbm.at[b, pl.ds(0, NSPAN // 2), pl.ds(d0, DB)], sem)
    span_half(NSPAN // 2, NSPAN)
    out_cp1 = pltpu.async_copy(
        o_v.at[pl.ds(NSPAN // 2, NSPAN // 2), :],
        out_hbm.at[b, pl.ds(NSPAN // 2, NSPAN // 2), pl.ds(d0, DB)], sem)
    out_cp0.wait()
    out_cp1.wait()


def kernel(sequence_tensor, span_indices):
    sp_flat = span_indices.astype(jnp.int32).reshape(-1)
    return _span_avg(sequence_tensor, sp_flat)


# speedup vs baseline: 1.0240x; 1.0240x over previous
import functools

import jax
import jax.numpy as jnp
from jax import lax
from jax.experimental import pallas as pl
from jax.experimental.pallas import tpu as pltpu
from jax.experimental.pallas import tpu_sc as plsc

B = 4
S = 2048
D = 1024
NSPAN = 128
SMAX = 64
EROWS = 2 * SMAX
L = 16
NC = 2
NS = 16
NW = NC * NS
DB = 128
NDB = D // DB

_mesh = plsc.VectorSubcoreMesh(core_axis_name="c", subcore_axis_name="s")


@functools.partial(
    pl.kernel,
    out_type=jax.ShapeDtypeStruct((B, NSPAN, D), jnp.float32),
    mesh=_mesh,
    compiler_params=pltpu.CompilerParams(
        needs_layout_passes=False, skip_device_barrier=True),
    scratch_types=[
        pltpu.VMEM((SMAX, DB), jnp.float32),
        pltpu.VMEM((EROWS, DB), jnp.float32),
        pltpu.VMEM((B * NSPAN * 2,), jnp.int32),
        pltpu.VMEM((NSPAN, DB), jnp.float32),
        pltpu.VMEM((NSPAN + L,), jnp.int32),
        pltpu.VMEM((NSPAN + L,), jnp.int32),
        pltpu.VMEM((NSPAN + L,), jnp.float32),
        pltpu.SemaphoreType.DMA,
    ],
)
def _span_avg(seq_hbm, sp_hbm, out_hbm, s_v, e_v, sp_v, o_v,
              hi_v, lo_v, iw_v, sem):
    wid = lax.axis_index("s") * NC + lax.axis_index("c")
    b = wid // NDB
    d0 = (wid % NDB) * DB
    seq_cp = pltpu.async_copy(
        seq_hbm.at[b, pl.ds(0, SMAX), pl.ds(d0, DB)], s_v, sem)
    pltpu.sync_copy(sp_hbm, sp_v)

    lanes = lax.iota(jnp.int32, L)

    waccs = [jnp.zeros((L,), jnp.int32) for _ in range(4)]
    for c in range(B * NSPAN // L):
        f = c * L + lanes
        sv = plsc.load_gather(sp_v, [2 * f])
        ev = plsc.load_gather(sp_v, [2 * f + 1])
        waccs[c % 4] = jnp.maximum(waccs[c % 4], ev - sv)
    w_glob = jnp.max(jnp.maximum(jnp.maximum(waccs[0], waccs[1]),
                                 jnp.maximum(waccs[2], waccs[3])))

    for c in range(NSPAN // L):
        f = (b * NSPAN + c * L) + lanes
        sv = plsc.load_gather(sp_v, [2 * f])
        ev = plsc.load_gather(sp_v, [2 * f + 1])
        weff = jnp.where(sv < ev, ev - sv, w_glob)
        hi = ev + SMAX
        lo = hi - weff
        invw = jnp.where(weff > 0, 1.0, 0.0) / jnp.maximum(
            weff, 1).astype(jnp.float32)
        sl = pl.ds(c * L, L)
        hi_v[sl] = hi
        lo_v[sl] = lo
        iw_v[sl] = invw

    seq_cp.wait()

    s0s = tuple(s_v[0, pl.ds(ch * L, L)] for ch in range(DB // L))
    zeros8 = tuple(jnp.zeros((L,), jnp.float32) for _ in range(DB // L))

    @plsc.parallel_loop(0, SMAX, step=1, unroll=4, carry=zeros8)
    def mids(m, accs):
        for ch in range(DB // L):
            e_v[m, pl.ds(ch * L, L)] = accs[ch]
        return tuple(a + s0 for a, s0 in zip(accs, s0s))

    @plsc.parallel_loop(0, SMAX, step=1, unroll=4, carry=mids)
    def _(k, accs):
        for ch in range(DB // L):
            e_v[SMAX + k, pl.ds(ch * L, L)] = accs[ch]
        return tuple(a + s_v[k, pl.ds(ch * L, L)]
                     for ch, a in enumerate(accs))

    @plsc.parallel_loop(0, NSPAN, step=1, unroll=8)
    def _(i):
        hi_j = hi_v[pl.ds(i, L)][0]
        lo_j = lo_v[pl.ds(i, L)][0]
        iw_j = iw_v[pl.ds(i, L)][0]
        for ch in range(DB // L):
            sl = pl.ds(ch * L, L)
            o_v[i, sl] = (e_v[hi_j, sl] - e_v[lo_j, sl]) * iw_j
        return ()

    pltpu.sync_copy(o_v, out_hbm.at[b, :, pl.ds(d0, DB)])


def kernel(sequence_tensor, span_indices):
    sp_flat = span_indices.astype(jnp.int32).reshape(-1)
    return _span_avg(sequence_tensor, sp_flat)
